# Initial kernel scaffold; baseline (speedup 1.0000x reference)
import jax
import jax.numpy as jnp
from jax.experimental import pallas as pl


def kernel(content_idxs, bert_word_embed):
    raise NotImplementedError("write your pallas kernel here")


# speedup vs baseline: 1.8799x; 1.8799x over previous
import functools

import jax
import jax.numpy as jnp
from jax import lax
from jax.experimental import pallas as pl
from jax.experimental.pallas import tpu as pltpu
from jax.experimental.pallas import tpu_sc as plsc

HIDDEN = 768
NC = 2
NS = 16
NW = NC * NS
LANES = 16
CHUNK = 64


def _zero_pad_rows(idx_v, buf, start, zeros16):
  for grp in range(CHUNK // LANES):
    v = idx_v[pl.ds(start + grp * LANES, LANES)]
    for lane in range(LANES):
      row = grp * LANES + lane

      @pl.when(v[lane] == 0)
      def _(row=row):
        def zcol(j, carry):
          buf[row, pl.ds(j * LANES, LANES)] = zeros16
          return carry

        lax.fori_loop(0, HIDDEN // LANES, zcol, jnp.int32(0))


def _make_gather(B):
  b_per_w = B // NW
  nchunks = b_per_w // CHUNK
  mesh = plsc.VectorSubcoreMesh(core_axis_name="c", subcore_axis_name="s")

  @functools.partial(
      pl.kernel,
      mesh=mesh,
      out_type=jax.ShapeDtypeStruct((B, HIDDEN), jnp.float32),
      scratch_types=[
          pltpu.VMEM((b_per_w,), jnp.int32),
          pltpu.VMEM((CHUNK, HIDDEN), jnp.float32),
          pltpu.VMEM((CHUNK, HIDDEN), jnp.float32),
          pltpu.SemaphoreType.DMA,
          pltpu.SemaphoreType.DMA,
      ],
  )
  def gather_kernel(table_hbm, idx_hbm, out_hbm, idx_v, buf0, buf1, sem0,
                    sem1):
    wid = lax.axis_index("s") * NC + lax.axis_index("c")
    base = wid * b_per_w
    bufs = (buf0, buf1)
    sems = (sem0, sem1)
    zeros16 = jnp.zeros((LANES,), jnp.float32)

    pltpu.sync_copy(idx_hbm.at[pl.ds(base, b_per_w)], idx_v)

    def gather_start(g, b):
      pltpu.make_async_copy(
          table_hbm.at[idx_v.at[pl.ds(g * CHUNK, CHUNK)]], bufs[b],
          sems[b]).start()

    def gather_wait(g, b):
      pltpu.make_async_copy(
          table_hbm.at[idx_v.at[pl.ds(g * CHUNK, CHUNK)]], bufs[b],
          sems[b]).wait()

    def process(g, b, prefetch):
      gather_wait(g, b)
      _zero_pad_rows(idx_v, bufs[b], g * CHUNK, zeros16)
      pltpu.sync_copy(bufs[b], out_hbm.at[pl.ds(base + g * CHUNK, CHUNK)])
      if prefetch:
        gather_start(g + 2, b)

    gather_start(0, 0)
    gather_start(1, 1)

    def outer(i, carry):
      process(2 * i, 0, True)
      process(2 * i + 1, 1, True)
      return carry

    lax.fori_loop(0, nchunks // 2 - 1, outer, jnp.int32(0))
    process(nchunks - 2, 0, False)
    process(nchunks - 1, 1, False)

  return gather_kernel


_gather = _make_gather(1024 * 200)


def kernel(content_idxs, bert_word_embed):
  idx = content_idxs.reshape(-1).astype(jnp.int32)
  out = _gather(bert_word_embed.astype(jnp.float32), idx)
  return out.reshape(content_idxs.shape + (HIDDEN,))


# speedup vs baseline: 1.8868x; 1.0037x over previous
import functools

import jax
import jax.numpy as jnp
from jax import lax
from jax.experimental import pallas as pl
from jax.experimental.pallas import tpu as pltpu
from jax.experimental.pallas import tpu_sc as plsc

HIDDEN = 768
NC = 2
NS = 16
NW = NC * NS
LANES = 16
CHUNK = 64


def _zero_pad_rows(idx_v, buf, start, zeros16):
  mm = idx_v[pl.ds(start, LANES)]
  for grp in range(1, CHUNK // LANES):
    mm = jnp.minimum(mm, idx_v[pl.ds(start + grp * LANES, LANES)])
  m = mm[0]
  for lane in range(1, LANES):
    m = jnp.minimum(m, mm[lane])

  @pl.when(m == 0)
  def _():
    for grp in range(CHUNK // LANES):
      v = idx_v[pl.ds(start + grp * LANES, LANES)]
      for lane in range(LANES):
        row = grp * LANES + lane

        @pl.when(v[lane] == 0)
        def _(row=row):
          def zcol(j, carry):
            buf[row, pl.ds(j * LANES, LANES)] = zeros16
            return carry

          lax.fori_loop(0, HIDDEN // LANES, zcol, jnp.int32(0))


def _make_gather(B):
  b_per_w = B // NW
  nchunks = b_per_w // CHUNK
  mesh = plsc.VectorSubcoreMesh(core_axis_name="c", subcore_axis_name="s")

  @functools.partial(
      pl.kernel,
      mesh=mesh,
      out_type=jax.ShapeDtypeStruct((B, HIDDEN), jnp.float32),
      scratch_types=[
          pltpu.VMEM((b_per_w,), jnp.int32),
          pltpu.VMEM((CHUNK, HIDDEN), jnp.float32),
          pltpu.VMEM((CHUNK, HIDDEN), jnp.float32),
          pltpu.SemaphoreType.DMA,
          pltpu.SemaphoreType.DMA,
      ],
  )
  def gather_kernel(table_hbm, idx_hbm, out_hbm, idx_v, buf0, buf1, sem0,
                    sem1):
    wid = lax.axis_index("s") * NC + lax.axis_index("c")
    base = wid * b_per_w
    bufs = (buf0, buf1)
    sems = (sem0, sem1)
    zeros16 = jnp.zeros((LANES,), jnp.float32)

    pltpu.sync_copy(idx_hbm.at[pl.ds(base, b_per_w)], idx_v)

    def gather_start(g, b):
      pltpu.make_async_copy(
          table_hbm.at[idx_v.at[pl.ds(g * CHUNK, CHUNK)]], bufs[b],
          sems[b]).start()

    def gather_wait(g, b):
      pltpu.make_async_copy(
          table_hbm.at[idx_v.at[pl.ds(g * CHUNK, CHUNK)]], bufs[b],
          sems[b]).wait()

    def process(g, b, prefetch):
      gather_wait(g, b)
      _zero_pad_rows(idx_v, bufs[b], g * CHUNK, zeros16)
      pltpu.sync_copy(bufs[b], out_hbm.at[pl.ds(base + g * CHUNK, CHUNK)])
      if prefetch:
        gather_start(g + 2, b)

    gather_start(0, 0)
    gather_start(1, 1)

    def outer(i, carry):
      process(2 * i, 0, True)
      process(2 * i + 1, 1, True)
      return carry

    lax.fori_loop(0, nchunks // 2 - 1, outer, jnp.int32(0))
    process(nchunks - 2, 0, False)
    process(nchunks - 1, 1, False)

  return gather_kernel


_gather = _make_gather(1024 * 200)


def kernel(content_idxs, bert_word_embed):
  idx = content_idxs.reshape(-1).astype(jnp.int32)
  out = _gather(bert_word_embed.astype(jnp.float32), idx)
  return out.reshape(content_idxs.shape + (HIDDEN,))
